# baseline (speedup 1.0000x reference)
import functools

import jax
import jax.numpy as jnp
from jax import lax
from jax.experimental import pallas as pl
from jax.experimental.pallas import tpu as pltpu
from jax.experimental.pallas import tpu_sc as plsc

_D = 32
_B = 16384

_INFO = plsc.get_sparse_core_info()
_NW = _INFO.num_cores * _INFO.num_subcores
_B_PER_W = _B // _NW
_CHUNK = 128
_N_CHUNKS = _B_PER_W // _CHUNK

_MESH = plsc.VectorSubcoreMesh(core_axis_name="c", subcore_axis_name="s")


@functools.partial(
    pl.kernel,
    mesh=_MESH,
    out_type=jax.ShapeDtypeStruct((_B, _D), jnp.float32),
    scratch_types=[
        pltpu.VMEM((_B_PER_W,), jnp.int32),
        pltpu.VMEM((_B_PER_W, _D), jnp.float32),
        pltpu.SemaphoreType.DMA,
    ],
    compiler_params=pltpu.CompilerParams(use_tc_tiling_on_sc=False),
)
def _gather_kernel(table_hbm, idx_hbm, out_hbm, idx_v, rows_v, sem):
    wid = lax.axis_index("s") * _INFO.num_cores + lax.axis_index("c")
    base = wid * _B_PER_W
    pltpu.sync_copy(idx_hbm.at[pl.ds(base, _B_PER_W)], idx_v)
    copies = []
    for j in range(_N_CHUNKS):
        sl = pl.ds(j * _CHUNK, _CHUNK)
        copies.append(
            pltpu.async_copy(table_hbm.at[idx_v.at[sl]], rows_v.at[sl], sem)
        )
    for c in copies:
        c.wait()
    pltpu.sync_copy(rows_v, out_hbm.at[pl.ds(base, _B_PER_W)])


def kernel(instance_ids, embedding_instance_weight):
    ids = instance_ids.astype(jnp.int32)
    return _gather_kernel(embedding_instance_weight, ids)


# speedup vs baseline: 3.8883x; 3.8883x over previous
import functools

import jax
import jax.numpy as jnp
from jax import lax
from jax.experimental import pallas as pl
from jax.experimental.pallas import tpu as pltpu
from jax.experimental.pallas import tpu_sc as plsc

_D = 32
_B = 16384
_V = 1000000

_INFO = plsc.get_sparse_core_info()
_NW = _INFO.num_cores * _INFO.num_subcores
_B_PER_W = _B // _NW
_MB = 16

_MESH = plsc.VectorSubcoreMesh(core_axis_name="c", subcore_axis_name="s")


@functools.partial(
    pl.kernel,
    mesh=_MESH,
    out_type=jax.ShapeDtypeStruct((_D, _B), jnp.float32),
    scratch_types=[
        pltpu.VMEM((_B_PER_W,), jnp.int32),
        pltpu.VMEM((_MB, _D, 128), jnp.float32),
        pltpu.VMEM((_D, _B_PER_W), jnp.float32),
        pltpu.SemaphoreType.DMA,
    ],
    compiler_params=pltpu.CompilerParams(needs_layout_passes=False),
)
def _gather_kernel(table_hbm, idx_hbm, out_hbm, idx_v, sb_v, rows_v, sem):
    wid = lax.axis_index("s") * _INFO.num_cores + lax.axis_index("c")
    base = wid * _B_PER_W
    pltpu.sync_copy(idx_hbm.at[pl.ds(base, _B_PER_W)], idx_v)
    iota16 = lax.iota(jnp.int32, 16)

    def group(g, _):
        jbase = g * _MB
        v16 = idx_v[pl.ds(jbase, _MB)]
        vtile = v16 >> jnp.int32(7)
        vlane = v16 & jnp.int32(127)
        copies = []
        for k in range(_MB):
            copies.append(
                pltpu.async_copy(
                    table_hbm.at[:, pl.ds(vtile[k] * 128, 128)],
                    sb_v.at[k],
                    sem,
                )
            )
        for c in copies:
            c.wait()
        for r in range(_D):
            vals = plsc.load_gather(
                sb_v,
                [iota16, jnp.full((16,), r, jnp.int32), vlane],
            )
            rows_v[r, pl.ds(jbase, _MB)] = vals
        return 0

    lax.fori_loop(0, _B_PER_W // _MB, group, 0)
    pltpu.sync_copy(rows_v, out_hbm.at[:, pl.ds(base, _B_PER_W)])


def kernel(instance_ids, embedding_instance_weight):
    ids = instance_ids.astype(jnp.int32)
    tab_t = embedding_instance_weight.T
    out_t = _gather_kernel(tab_t, ids)
    return out_t.T


# speedup vs baseline: 3.9281x; 1.0102x over previous
import functools

import jax
import jax.numpy as jnp
from jax import lax
from jax.experimental import pallas as pl
from jax.experimental.pallas import tpu as pltpu
from jax.experimental.pallas import tpu_sc as plsc

_D = 32
_B = 16384
_V = 1000000

_INFO = plsc.get_sparse_core_info()
_NW = _INFO.num_cores * _INFO.num_subcores
_B_PER_W = _B // _NW
_MB = 16

_MESH = plsc.VectorSubcoreMesh(core_axis_name="c", subcore_axis_name="s")


@functools.partial(
    pl.kernel,
    mesh=_MESH,
    out_type=jax.ShapeDtypeStruct((_D, _B), jnp.float32),
    scratch_types=[
        pltpu.VMEM((_B_PER_W,), jnp.int32),
        pltpu.VMEM((_MB, _D, 128), jnp.float32),
        pltpu.VMEM((_D, _B_PER_W), jnp.float32),
        pltpu.SemaphoreType.DMA,
    ],
    compiler_params=pltpu.CompilerParams(needs_layout_passes=False),
)
def _gather_kernel(table_hbm, idx_hbm, out_hbm, idx_v, sb_v, rows_v, sem):
    wid = lax.axis_index("s") * _INFO.num_cores + lax.axis_index("c")
    base = wid * _B_PER_W
    pltpu.sync_copy(idx_hbm.at[pl.ds(base, _B_PER_W)], idx_v)
    iota16 = lax.iota(jnp.int32, 16)

    def group(g, _):
        jbase = g * _MB
        v16 = idx_v[pl.ds(jbase, _MB)]
        vtile = v16 >> jnp.int32(7)
        vlane = v16 & jnp.int32(127)
        copies = []
        for k in range(_MB):
            copies.append(
                pltpu.async_copy(
                    table_hbm.at[:, pl.ds(vtile[k] * 128, 128)],
                    sb_v.at[k],
                    sem,
                )
            )
        for k in range(_MB):
            copies[k].wait()
            lane16 = jnp.full((16,), vlane[k], jnp.int32)
            col16 = jnp.full((16,), jbase + k, jnp.int32)
            for half in range(2):
                rsel = iota16 + jnp.int32(half * 16)
                vals = plsc.load_gather(sb_v.at[k], [rsel, lane16])
                plsc.store_scatter(rows_v, [rsel, col16], vals)
        return 0

    lax.fori_loop(0, _B_PER_W // _MB, group, 0)
    pltpu.sync_copy(rows_v, out_hbm.at[:, pl.ds(base, _B_PER_W)])


def kernel(instance_ids, embedding_instance_weight):
    ids = instance_ids.astype(jnp.int32)
    tab_t = embedding_instance_weight.T
    out_t = _gather_kernel(tab_t, ids)
    return out_t.T


# speedup vs baseline: 3.9297x; 1.0004x over previous
import functools

import jax
import jax.numpy as jnp
from jax import lax
from jax.experimental import pallas as pl
from jax.experimental.pallas import tpu as pltpu
from jax.experimental.pallas import tpu_sc as plsc

_D = 32
_B = 16384

_INFO = plsc.get_sparse_core_info()
_NW = _INFO.num_cores * _INFO.num_subcores
_B_PER_W = _B // _NW
_MB = 16

_MESH = plsc.VectorSubcoreMesh(core_axis_name="c", subcore_axis_name="s")


@functools.partial(
    pl.kernel,
    mesh=_MESH,
    out_type=jax.ShapeDtypeStruct((_D, _B), jnp.float32),
    scratch_types=[
        pltpu.VMEM((_B_PER_W,), jnp.int32),
        pltpu.VMEM((_MB, _D, 128), jnp.float32),
        pltpu.VMEM((_D, _B_PER_W), jnp.float32),
        pltpu.SemaphoreType.DMA,
    ],
    compiler_params=pltpu.CompilerParams(needs_layout_passes=False),
)
def _gather_kernel(table_hbm, idx_hbm, out_hbm, idx_v, sb_v, rows_v, sem):
    wid = lax.axis_index("s") * _INFO.num_cores + lax.axis_index("c")
    base = wid * _B_PER_W
    pltpu.sync_copy(idx_hbm.at[pl.ds(base, _B_PER_W)], idx_v)
    iota16 = lax.iota(jnp.int32, 16)

    def group(g, _):
        jbase = g * _MB
        v16 = idx_v[pl.ds(jbase, _MB)]
        vtile = v16 >> jnp.int32(7)
        vlane = v16 & jnp.int32(127)
        copies = []
        for k in range(_MB):
            copies.append(
                pltpu.async_copy(
                    table_hbm.at[:, pl.ds(vtile[k] * 128, 128)],
                    sb_v.at[k],
                    sem,
                )
            )
        for k in range(_MB):
            copies[k].wait()
            lane16 = jnp.full((16,), vlane[k], jnp.int32)
            col16 = jnp.full((16,), jbase + k, jnp.int32)
            for half in range(2):
                rsel = iota16 + jnp.int32(half * 16)
                vals = plsc.load_gather(sb_v.at[k], [rsel, lane16])
                plsc.store_scatter(rows_v, [rsel, col16], vals)
        return 0

    lax.fori_loop(0, _B_PER_W // _MB, group, 0)
    pltpu.sync_copy(rows_v, out_hbm.at[:, pl.ds(base, _B_PER_W)])


def kernel(instance_ids, embedding_instance_weight):
    ids = instance_ids.astype(jnp.int32)
    tab_t = embedding_instance_weight.T
    out_t = _gather_kernel(tab_t, ids)
    return out_t.T
